# Initial kernel scaffold; baseline (speedup 1.0000x reference)
import jax
import jax.numpy as jnp
from jax.experimental import pallas as pl


def kernel(x, edge_index, W1_l, W1_r, b1, g1, be1, W2_l, W2_r, b2, g2, be2):
    raise NotImplementedError("write your pallas kernel here")


# speedup vs baseline: 5.4471x; 5.4471x over previous
import functools

import jax
import jax.numpy as jnp
from jax import lax
from jax.experimental import pallas as pl
from jax.experimental.pallas import tpu as pltpu
from jax.experimental.pallas import tpu_sc as plsc

N = 10000
D_HID = 64
DEG_W = 16
NC = 2
NS = 16
NW = NC * NS
CHUNK = 128
N_PAD = 10240
RPS = N_PAD // NS
EPS = 1e-5


def _sc_mesh():
    return plsc.VectorSubcoreMesh(core_axis_name="c", subcore_axis_name="s")


def _zero_rows(ref, n_rows, width):
    zero16 = jnp.zeros((16,), jnp.float32)

    def body(i, _):
        for j in range(width // 16):
            ref[i, pl.ds(j * 16, 16)] = zero16
        return 0

    lax.fori_loop(0, n_rows, body, 0)


def _sc_aggregate(p, src, dst, ch_per_tile, with_deg):
    out_type = [jax.ShapeDtypeStruct((2 * N_PAD, D_HID), jnp.float32)]
    scratch = [
        pltpu.VMEM((CHUNK,), jnp.int32),
        pltpu.VMEM((CHUNK,), jnp.int32),
        pltpu.VMEM((CHUNK, D_HID), jnp.float32),
        pltpu.VMEM((RPS, D_HID), jnp.float32),
        pltpu.VMEM_SHARED((N_PAD, D_HID), jnp.float32),
        pltpu.SemaphoreType.DMA,
    ]
    if with_deg:
        out_type.append(jax.ShapeDtypeStruct((2 * N_PAD, DEG_W), jnp.float32))
        scratch += [
            pltpu.VMEM((CHUNK, DEG_W), jnp.float32),
            pltpu.VMEM((RPS, DEG_W), jnp.float32),
            pltpu.VMEM_SHARED((N_PAD, DEG_W), jnp.float32),
        ]

    def body(p_hbm, src_hbm, dst_hbm, *rest):
        if with_deg:
            (acc_out, deg_out,
             src_v, dst_v, rows_v, zacc_v, acc_sh, sem,
             ones_v, zdeg_v, deg_sh) = rest
        else:
            (acc_out,
             src_v, dst_v, rows_v, zacc_v, acc_sh, sem) = rest

        cid = lax.axis_index("c")
        sid = lax.axis_index("s")

        _zero_rows(zacc_v, RPS, D_HID)
        row0 = sid * RPS
        pltpu.sync_copy(zacc_v, acc_sh.at[pl.ds(row0, RPS)])
        if with_deg:
            _zero_rows(zdeg_v, RPS, DEG_W)
            pltpu.sync_copy(zdeg_v, deg_sh.at[pl.ds(row0, RPS)])
            one16 = jnp.ones((16,), jnp.float32)

            def fill_ones(i, _):
                ones_v[i, :] = one16
                return 0

            lax.fori_loop(0, CHUNK, fill_ones, 0)
        plsc.subcore_barrier()

        base = (cid * NS + sid) * ch_per_tile * CHUNK

        def edge_chunk(c, _):
            off = base + c * CHUNK
            pltpu.sync_copy(src_hbm.at[pl.ds(off, CHUNK)], src_v)
            pltpu.sync_copy(dst_hbm.at[pl.ds(off, CHUNK)], dst_v)
            pltpu.async_copy(p_hbm.at[src_v], rows_v, sem).wait()
            pltpu.sync_copy(rows_v, acc_sh.at[dst_v], add=True)
            if with_deg:
                pltpu.sync_copy(ones_v, deg_sh.at[dst_v], add=True)
            return 0

        lax.fori_loop(0, ch_per_tile, edge_chunk, 0)
        plsc.subcore_barrier()

        out0 = cid * N_PAD + row0
        pltpu.sync_copy(acc_sh.at[pl.ds(row0, RPS)], acc_out.at[pl.ds(out0, RPS)])
        if with_deg:
            pltpu.sync_copy(deg_sh.at[pl.ds(row0, RPS)], deg_out.at[pl.ds(out0, RPS)])

    fn = pl.kernel(
        body,
        out_type=tuple(out_type),
        mesh=_sc_mesh(),
        scratch_types=tuple(scratch),
        compiler_params=pltpu.CompilerParams(use_tc_tiling_on_sc=False),
    )
    res = fn(p, src, dst)
    if with_deg:
        return res
    return res[0] if isinstance(res, (tuple, list)) else res


def _project2(x, wl, wr):
    def body(x_ref, wl_ref, wr_ref, p_ref, r_ref):
        xv = x_ref[...]
        p_ref[...] = jnp.dot(xv, wl_ref[...], preferred_element_type=jnp.float32)
        r_ref[...] = jnp.dot(xv, wr_ref[...], preferred_element_type=jnp.float32)

    d = wl.shape[1]
    return pl.pallas_call(
        body,
        out_shape=[jax.ShapeDtypeStruct((N, d), jnp.float32)] * 2,
    )(x, wl, wr)


def _deg_from_parts(deg_ref):
    d = deg_ref[0:N, :] + deg_ref[N_PAD:N_PAD + N, :]
    return jnp.max(d, axis=1, keepdims=True)


def _bn_relu(pre, g, be):
    mu = jnp.mean(pre, axis=0, keepdims=True)
    var = jnp.mean((pre - mu) ** 2, axis=0, keepdims=True)
    h = g * (pre - mu) * lax.rsqrt(var + EPS) + be
    return jnp.maximum(h, 0.0)


def _layer1_post(acc, deg, r, b1, g1, be1):
    def body(acc_ref, deg_ref, r_ref, b_ref, g_ref, be_ref, h_ref):
        a = acc_ref[0:N, :] + acc_ref[N_PAD:N_PAD + N, :]
        degv = _deg_from_parts(deg_ref)
        pre = a / jnp.maximum(degv, 1.0) + r_ref[...] + b_ref[...]
        h_ref[...] = _bn_relu(pre, g_ref[...], be_ref[...])

    return pl.pallas_call(
        body,
        out_shape=jax.ShapeDtypeStruct((N, D_HID), jnp.float32),
    )(acc, deg, r, b1, g1, be1)


def _layer2_post(acc, deg, h, wl, wr, b2, g2, be2):
    def body(acc_ref, deg_ref, h_ref, wl_ref, wr_ref, b_ref, g_ref, be_ref, o_ref):
        a = acc_ref[0:N, :] + acc_ref[N_PAD:N_PAD + N, :]
        degv = _deg_from_parts(deg_ref)
        agg = a / jnp.maximum(degv, 1.0)
        z = (jnp.dot(agg, wl_ref[...], preferred_element_type=jnp.float32)
             + jnp.dot(h_ref[...], wr_ref[...], preferred_element_type=jnp.float32)
             + b_ref[...])
        o_ref[...] = _bn_relu(z, g_ref[...], be_ref[...])

    d_out = wl.shape[1]
    return pl.pallas_call(
        body,
        out_shape=jax.ShapeDtypeStruct((N, d_out), jnp.float32),
    )(acc, deg, h, wl, wr, b2, g2, be2)


def kernel(x, edge_index, W1_l, W1_r, b1, g1, be1, W2_l, W2_r, b2, g2, be2):
    ei = edge_index.astype(jnp.int32)
    n_edges = ei.shape[1]
    tile_quota = NW * CHUNK
    e_pad = ((n_edges + tile_quota - 1) // tile_quota) * tile_quota
    ch_per_tile = e_pad // (NW * CHUNK)
    pad = e_pad - n_edges
    src = jnp.concatenate([ei[0], jnp.zeros((pad,), jnp.int32)])
    dst = jnp.concatenate([ei[1], jnp.full((pad,), N, jnp.int32)])

    b1r, g1r, be1r = b1.reshape(1, -1), g1.reshape(1, -1), be1.reshape(1, -1)
    b2r, g2r, be2r = b2.reshape(1, -1), g2.reshape(1, -1), be2.reshape(1, -1)

    p1, r1 = _project2(x, W1_l, W1_r)
    acc1, deg = _sc_aggregate(p1, src, dst, ch_per_tile, with_deg=True)
    h = _layer1_post(acc1, deg, r1, b1r, g1r, be1r)
    acc2 = _sc_aggregate(h, src, dst, ch_per_tile, with_deg=False)
    return _layer2_post(acc2, deg, h, W2_l, W2_r, b2r, g2r, be2r)


# speedup vs baseline: 5.9534x; 1.0929x over previous
import functools

import jax
import jax.numpy as jnp
from jax import lax
from jax.experimental import pallas as pl
from jax.experimental.pallas import tpu as pltpu
from jax.experimental.pallas import tpu_sc as plsc

N = 10000
D_HID = 64
DEG_W = 16
NC = 2
NS = 16
NW = NC * NS
CHUNK = 128
NBUF = 5
N_PAD = 10240
RPS = N_PAD // NS
EPS = 1e-5


def _sc_mesh():
    return plsc.VectorSubcoreMesh(core_axis_name="c", subcore_axis_name="s")


def _zero_rows(ref, n_rows, width):
    zero16 = jnp.zeros((16,), jnp.float32)

    def body(i, _):
        for j in range(width // 16):
            ref[i, pl.ds(j * 16, 16)] = zero16
        return 0

    lax.fori_loop(0, n_rows, body, 0)


def _sc_aggregate(p, src, dst, ch_per_tile, with_deg):
    assert ch_per_tile % NBUF == 0

    out_type = [jax.ShapeDtypeStruct((2 * N_PAD, D_HID), jnp.float32)]
    scratch = [
        pltpu.VMEM((ch_per_tile, CHUNK), jnp.int32),
        pltpu.VMEM((ch_per_tile, CHUNK), jnp.int32),
    ]
    scratch += [pltpu.VMEM((CHUNK, D_HID), jnp.float32) for _ in range(NBUF)]
    scratch += [pltpu.SemaphoreType.DMA for _ in range(NBUF)]
    scratch.append(pltpu.VMEM_SHARED((N_PAD, D_HID), jnp.float32))
    if with_deg:
        out_type.append(jax.ShapeDtypeStruct((2 * N_PAD, DEG_W), jnp.float32))
        scratch += [
            pltpu.VMEM((CHUNK, DEG_W), jnp.float32),
            pltpu.VMEM_SHARED((N_PAD, DEG_W), jnp.float32),
        ]

    def body(p_hbm, src_hbm, dst_hbm, *rest):
        if with_deg:
            (acc_out, deg_out, src_v, dst_v, *bufs) = rest
            rows = bufs[:NBUF]
            sems = bufs[NBUF:2 * NBUF]
            acc_sh, ones_v, deg_sh = bufs[2 * NBUF:]
        else:
            (acc_out, src_v, dst_v, *bufs) = rest
            rows = bufs[:NBUF]
            sems = bufs[NBUF:2 * NBUF]
            (acc_sh,) = bufs[2 * NBUF:]

        cid = lax.axis_index("c")
        sid = lax.axis_index("s")
        row0 = sid * RPS

        blk0 = (cid * NS + sid) * ch_per_tile
        pltpu.sync_copy(src_hbm.at[pl.ds(blk0, ch_per_tile)], src_v)
        pltpu.sync_copy(dst_hbm.at[pl.ds(blk0, ch_per_tile)], dst_v)

        _zero_rows(rows[0], CHUNK, D_HID)
        for k in range(RPS // CHUNK):
            pltpu.sync_copy(rows[0], acc_sh.at[pl.ds(row0 + k * CHUNK, CHUNK)])
        if with_deg:
            _zero_rows(ones_v, CHUNK, DEG_W)
            for k in range(RPS // CHUNK):
                pltpu.sync_copy(ones_v, deg_sh.at[pl.ds(row0 + k * CHUNK, CHUNK)])
            one16 = jnp.ones((16,), jnp.float32)

            def fill_ones(i, _):
                ones_v[i, :] = one16
                return 0

            lax.fori_loop(0, CHUNK, fill_ones, 0)
        plsc.subcore_barrier()

        def group(i, _):
            c0 = i * NBUF
            copies = [
                pltpu.async_copy(p_hbm.at[src_v.at[c0 + b]], rows[b], sems[b])
                for b in range(NBUF)
            ]
            for b in range(NBUF):
                copies[b].wait()
                pltpu.sync_copy(rows[b], acc_sh.at[dst_v.at[c0 + b]], add=True)
                if with_deg:
                    pltpu.sync_copy(ones_v, deg_sh.at[dst_v.at[c0 + b]], add=True)
            return 0

        lax.fori_loop(0, ch_per_tile // NBUF, group, 0)
        plsc.subcore_barrier()

        out0 = cid * N_PAD + row0
        pltpu.sync_copy(acc_sh.at[pl.ds(row0, RPS)], acc_out.at[pl.ds(out0, RPS)])
        if with_deg:
            pltpu.sync_copy(deg_sh.at[pl.ds(row0, RPS)], deg_out.at[pl.ds(out0, RPS)])

    fn = pl.kernel(
        body,
        out_type=tuple(out_type),
        mesh=_sc_mesh(),
        scratch_types=tuple(scratch),
        compiler_params=pltpu.CompilerParams(use_tc_tiling_on_sc=False),
    )
    res = fn(p, src, dst)
    if with_deg:
        return res
    return res[0] if isinstance(res, (tuple, list)) else res


def _project2(x, wl, wr):
    def body(x_ref, wl_ref, wr_ref, p_ref, r_ref):
        xv = x_ref[...]
        p_ref[...] = jnp.dot(xv, wl_ref[...], preferred_element_type=jnp.float32)
        r_ref[...] = jnp.dot(xv, wr_ref[...], preferred_element_type=jnp.float32)

    d = wl.shape[1]
    return pl.pallas_call(
        body,
        out_shape=[jax.ShapeDtypeStruct((N, d), jnp.float32)] * 2,
    )(x, wl, wr)


def _deg_from_parts(deg_ref):
    d = deg_ref[0:N, :] + deg_ref[N_PAD:N_PAD + N, :]
    return jnp.max(d, axis=1, keepdims=True)


def _bn_relu(pre, g, be):
    mu = jnp.mean(pre, axis=0, keepdims=True)
    var = jnp.mean((pre - mu) ** 2, axis=0, keepdims=True)
    h = g * (pre - mu) * lax.rsqrt(var + EPS) + be
    return jnp.maximum(h, 0.0)


def _layer1_post(acc, deg, r, b1, g1, be1):
    def body(acc_ref, deg_ref, r_ref, b_ref, g_ref, be_ref, h_ref):
        a = acc_ref[0:N, :] + acc_ref[N_PAD:N_PAD + N, :]
        degv = _deg_from_parts(deg_ref)
        pre = a / jnp.maximum(degv, 1.0) + r_ref[...] + b_ref[...]
        h_ref[...] = _bn_relu(pre, g_ref[...], be_ref[...])

    return pl.pallas_call(
        body,
        out_shape=jax.ShapeDtypeStruct((N, D_HID), jnp.float32),
    )(acc, deg, r, b1, g1, be1)


def _layer2_post(acc, deg, h, wl, wr, b2, g2, be2):
    def body(acc_ref, deg_ref, h_ref, wl_ref, wr_ref, b_ref, g_ref, be_ref, o_ref):
        a = acc_ref[0:N, :] + acc_ref[N_PAD:N_PAD + N, :]
        degv = _deg_from_parts(deg_ref)
        agg = a / jnp.maximum(degv, 1.0)
        z = (jnp.dot(agg, wl_ref[...], preferred_element_type=jnp.float32)
             + jnp.dot(h_ref[...], wr_ref[...], preferred_element_type=jnp.float32)
             + b_ref[...])
        o_ref[...] = _bn_relu(z, g_ref[...], be_ref[...])

    d_out = wl.shape[1]
    return pl.pallas_call(
        body,
        out_shape=jax.ShapeDtypeStruct((N, d_out), jnp.float32),
    )(acc, deg, h, wl, wr, b2, g2, be2)


def kernel(x, edge_index, W1_l, W1_r, b1, g1, be1, W2_l, W2_r, b2, g2, be2):
    ei = edge_index.astype(jnp.int32)
    n_edges = ei.shape[1]
    tile_quota = NW * CHUNK * NBUF
    e_pad = ((n_edges + tile_quota - 1) // tile_quota) * tile_quota
    ch_per_tile = e_pad // (NW * CHUNK)
    pad = e_pad - n_edges
    src = jnp.concatenate([ei[0], jnp.zeros((pad,), jnp.int32)])
    dst = jnp.concatenate([ei[1], jnp.full((pad,), N, jnp.int32)])
    src = src.reshape(NW * ch_per_tile, CHUNK)
    dst = dst.reshape(NW * ch_per_tile, CHUNK)

    b1r, g1r, be1r = b1.reshape(1, -1), g1.reshape(1, -1), be1.reshape(1, -1)
    b2r, g2r, be2r = b2.reshape(1, -1), g2.reshape(1, -1), be2.reshape(1, -1)

    p1, r1 = _project2(x, W1_l, W1_r)
    acc1, deg = _sc_aggregate(p1, src, dst, ch_per_tile, with_deg=True)
    h = _layer1_post(acc1, deg, r1, b1r, g1r, be1r)
    acc2 = _sc_aggregate(h, src, dst, ch_per_tile, with_deg=False)
    return _layer2_post(acc2, deg, h, W2_l, W2_r, b2r, g2r, be2r)
